# baseline (speedup 1.0000x reference)
import functools

import jax
import jax.numpy as jnp
from jax import lax
from jax.experimental import pallas as pl
from jax.experimental.pallas import tpu as pltpu
from jax.experimental.pallas import tpu_sc as plsc

N = 10000
NP = 10240
E = 320000
K = 32
ERP = 10240
EP = ERP * K
RTE = ERP // 32
TROWS = NP // 16
BR = 1024
GN = NP // BR
RNG = 4
LK = 2
RNGD = 4
LKD = 2

_MESH = plsc.VectorSubcoreMesh(core_axis_name="c", subcore_axis_name="s")


def _deg(dstp):
    @functools.partial(
        pl.kernel,
        out_type=jax.ShapeDtypeStruct((2 * NP,), jnp.float32),
        mesh=_MESH,
        compiler_params=pltpu.CompilerParams(use_tc_tiling_on_sc=False),
        scratch_types=[
            pltpu.VMEM_SHARED((NP,), jnp.float32),
            pltpu.VMEM((RTE, K), jnp.int32),
            pltpu.VMEM((K,), jnp.float32),
            pltpu.VMEM((TROWS,), jnp.float32),
            pltpu.SemaphoreType.DMA,
        ],
    )
    def deg_kernel(dst_hbm, out_hbm, acc, idxd, ones_v, zer_v, sem):
        c = lax.axis_index("c")
        s = lax.axis_index("s")
        w = s * 2 + c
        for i in range(K // 16):
            ones_v[pl.ds(16 * i, 16)] = jnp.ones((16,), jnp.float32)
        for i in range(TROWS // 16):
            zer_v[pl.ds(16 * i, 16)] = jnp.zeros((16,), jnp.float32)
        pltpu.sync_copy(zer_v, acc.at[pl.ds(s * TROWS, TROWS)])
        pltpu.sync_copy(dst_hbm.at[pl.ds(w * RTE, RTE)], idxd)
        plsc.subcore_barrier()

        def body(i, carry):
            for b in range(8):
                pltpu.async_copy(ones_v, acc.at[idxd.at[i * 8 + b]], sem,
                                 add=True)
            for b in range(8):
                pltpu.make_async_copy(ones_v, acc.at[idxd.at[i * 8 + b]],
                                      sem).wait()
            return carry

        lax.fori_loop(0, RTE // 8, body, 0)
        plsc.subcore_barrier()
        pltpu.sync_copy(acc.at[pl.ds(s * TROWS, TROWS)],
                        out_hbm.at[pl.ds(c * NP + s * TROWS, TROWS)])

    return deg_kernel(dstp)


def _agg(tbl, srcp, dstp):
    @functools.partial(
        pl.kernel,
        out_type=jax.ShapeDtypeStruct((2 * NP, 128), jnp.float32),
        mesh=_MESH,
        compiler_params=pltpu.CompilerParams(use_tc_tiling_on_sc=False),
        scratch_types=[
            pltpu.VMEM_SHARED((NP, 128), jnp.float32),
            pltpu.VMEM((RTE, K), jnp.int32),
            pltpu.VMEM((RTE, K), jnp.int32),
            pltpu.VMEM((RNG, K, 128), jnp.float32),
            pltpu.SemaphoreType.DMA((RNG,)),
            pltpu.SemaphoreType.DMA((RNG,)),
        ],
    )
    def agg_kernel(tbl_hbm, src_hbm, dst_hbm, out_hbm, acc, idxs, idxd, rows,
                   gsem, ssem):
        c = lax.axis_index("c")
        s = lax.axis_index("s")
        w = s * 2 + c
        rb = s * TROWS
        pltpu.sync_copy(tbl_hbm.at[pl.ds(rb, TROWS)], acc.at[pl.ds(rb, TROWS)])
        pltpu.sync_copy(src_hbm.at[pl.ds(w * RTE, RTE)], idxs)
        pltpu.sync_copy(dst_hbm.at[pl.ds(w * RTE, RTE)], idxd)
        plsc.subcore_barrier()

        def fire_g(j, r):
            pltpu.async_copy(tbl_hbm.at[idxs.at[j]], rows.at[r], gsem.at[r])

        def wait_g(j, r):
            pltpu.make_async_copy(tbl_hbm.at[idxs.at[j]], rows.at[r],
                                  gsem.at[r]).wait()

        def fire_s(j, r):
            pltpu.async_copy(rows.at[r], acc.at[idxd.at[j]], ssem.at[r],
                             add=True)

        def wait_s(j, r):
            pltpu.make_async_copy(rows.at[r], acc.at[idxd.at[j]],
                                  ssem.at[r]).wait()

        for r in range(LK):
            fire_g(r, r)

        def body(i, carry):
            for r in range(RNG):
                j = i * RNG + r
                rl = (r + LK) % RNG

                @pl.when(j + LK < RTE)
                def _():
                    @pl.when(j + LK >= RNG)
                    def _():
                        wait_s(j + LK - RNG, rl)
                    fire_g(j + LK, rl)

                wait_g(j, r)
                fire_s(j, r)
            return carry

        lax.fori_loop(0, RTE // RNG, body, 0)
        for r in range(RNG):
            wait_s(RTE - RNG + r, r)
        plsc.subcore_barrier()
        pltpu.sync_copy(acc.at[pl.ds(rb, TROWS)],
                        out_hbm.at[pl.ds(c * NP + rb, TROWS)])

    return agg_kernel(tbl, srcp, dstp)


def _dec(mu, srcp, dstp):
    @functools.partial(
        pl.kernel,
        out_type=jax.ShapeDtypeStruct((EP,), jnp.float32),
        mesh=_MESH,
        compiler_params=pltpu.CompilerParams(use_tc_tiling_on_sc=False,
                                             needs_layout_passes=False),
        scratch_types=[
            pltpu.VMEM((RTE, K), jnp.int32),
            pltpu.VMEM((RTE, K), jnp.int32),
            pltpu.VMEM((RNGD, K, 64), jnp.float32),
            pltpu.VMEM((RNGD, K, 64), jnp.float32),
            pltpu.VMEM((RTE * K,), jnp.float32),
            pltpu.SemaphoreType.DMA((RNGD,)),
            pltpu.SemaphoreType.DMA((RNGD,)),
        ],
    )
    def dec_kernel(mu_hbm, src_hbm, dst_hbm, out_hbm, idxs, idxd, zsv, zdv,
                   vout, gs_sem, gd_sem):
        c = lax.axis_index("c")
        s = lax.axis_index("s")
        w = s * 2 + c
        pltpu.sync_copy(src_hbm.at[pl.ds(w * RTE, RTE)], idxs)
        pltpu.sync_copy(dst_hbm.at[pl.ds(w * RTE, RTE)], idxd)
        lane = lax.iota(jnp.int32, 16)
        rowv = [lane + (g * 16) for g in range(K // 16)]

        def fire_g(j, r):
            pltpu.async_copy(mu_hbm.at[idxs.at[j]], zsv.at[r], gs_sem.at[r])
            pltpu.async_copy(mu_hbm.at[idxd.at[j]], zdv.at[r], gd_sem.at[r])

        def wait_g(j, r):
            pltpu.make_async_copy(mu_hbm.at[idxs.at[j]], zsv.at[r],
                                  gs_sem.at[r]).wait()
            pltpu.make_async_copy(mu_hbm.at[idxd.at[j]], zdv.at[r],
                                  gd_sem.at[r]).wait()

        for r in range(LKD):
            fire_g(r, r)

        def body(i, carry):
            for r in range(RNGD):
                j = i * RNGD + r
                rl = (r + LKD) % RNGD

                @pl.when(j + LKD < RTE)
                def _():
                    fire_g(j + LKD, rl)

                wait_g(j, r)
                zs2 = zsv.at[r]
                zd2 = zdv.at[r]

                def dbody(dd, accs):
                    out = list(accs)
                    for u in range(4):
                        d = dd * 4 + u
                        col = jnp.full((16,), d, jnp.int32)
                        for g in range(K // 16):
                            a = plsc.load_gather(zs2, [rowv[g], col])
                            b = plsc.load_gather(zd2, [rowv[g], col])
                            out[g] = out[g] + a * b
                    return tuple(out)

                zero = jnp.zeros((16,), jnp.float32)
                accs = lax.fori_loop(0, 16, dbody, (zero,) * (K // 16))
                for g in range(K // 16):
                    vout[pl.ds(j * K + g * 16, 16)] = (
                        1.0 / (1.0 + jnp.exp(-accs[g])))
            return carry

        lax.fori_loop(0, RTE // RNGD, body, 0)
        pltpu.sync_copy(vout, out_hbm.at[pl.ds(w * RTE * K, RTE * K)])

    return dec_kernel(mu, srcp, dstp)


def _tc_prep(xp, w1, degp):
    def body(x_ref, w_ref, d0_ref, d1_ref, dis_ref, o_ref):
        deg = d0_ref[0] + d1_ref[0] + 1.0
        dis = lax.rsqrt(deg)
        dis_ref[...] = dis
        o_ref[...] = jnp.dot(x_ref[...], w_ref[...],
                             preferred_element_type=jnp.float32) * dis

    return pl.pallas_call(
        body,
        grid=(GN,),
        in_specs=[
            pl.BlockSpec((BR, 128), lambda i: (i, 0)),
            pl.BlockSpec((128, 128), lambda i: (0, 0)),
            pl.BlockSpec((1, BR, 1), lambda i: (0, i, 0)),
            pl.BlockSpec((1, BR, 1), lambda i: (1, i, 0)),
        ],
        out_specs=[
            pl.BlockSpec((BR, 1), lambda i: (i, 0)),
            pl.BlockSpec((BR, 128), lambda i: (i, 0)),
        ],
        out_shape=[
            jax.ShapeDtypeStruct((NP, 1), jnp.float32),
            jax.ShapeDtypeStruct((NP, 128), jnp.float32),
        ],
    )(xp, w1, degp, degp)


def _tc_mid1(p, v, b, dis, w):
    def body(p_ref, v_ref, b_ref, dis_ref, w_ref, oa_ref, ob_ref):
        agg = p_ref[0] + p_ref[1] - v_ref[...]
        dis_v = dis_ref[...]
        h = jnp.maximum(agg * dis_v + b_ref[...], 0.0)
        o = jnp.dot(h, w_ref[...], preferred_element_type=jnp.float32) * dis_v
        oa_ref[...] = o[:, :128]
        ob_ref[...] = o[:, 128:]

    return pl.pallas_call(
        body,
        grid=(GN,),
        in_specs=[
            pl.BlockSpec((2, BR, 128), lambda i: (0, i, 0)),
            pl.BlockSpec((BR, 128), lambda i: (i, 0)),
            pl.BlockSpec((1, 128), lambda i: (0, 0)),
            pl.BlockSpec((BR, 1), lambda i: (i, 0)),
            pl.BlockSpec((128, 256), lambda i: (0, 0)),
        ],
        out_specs=[
            pl.BlockSpec((BR, 128), lambda i: (i, 0)),
            pl.BlockSpec((BR, 128), lambda i: (i, 0)),
        ],
        out_shape=[
            jax.ShapeDtypeStruct((NP, 128), jnp.float32),
            jax.ShapeDtypeStruct((NP, 128), jnp.float32),
        ],
    )(p, v, b, dis, w)


def _tc_mid2(pa, pb, va, vb, b, dis, w):
    def body(pa_ref, pb_ref, va_ref, vb_ref, b_ref, dis_ref, w_ref, o_ref):
        agg = jnp.concatenate(
            [pa_ref[0] + pa_ref[1] - va_ref[...],
             pb_ref[0] + pb_ref[1] - vb_ref[...]], axis=1)
        dis_v = dis_ref[...]
        h = jnp.maximum(agg * dis_v + b_ref[...], 0.0)
        o_ref[...] = jnp.dot(h, w_ref[...],
                             preferred_element_type=jnp.float32) * dis_v

    return pl.pallas_call(
        body,
        grid=(GN,),
        in_specs=[
            pl.BlockSpec((2, BR, 128), lambda i: (0, i, 0)),
            pl.BlockSpec((2, BR, 128), lambda i: (0, i, 0)),
            pl.BlockSpec((BR, 128), lambda i: (i, 0)),
            pl.BlockSpec((BR, 128), lambda i: (i, 0)),
            pl.BlockSpec((1, 256), lambda i: (0, 0)),
            pl.BlockSpec((BR, 1), lambda i: (i, 0)),
            pl.BlockSpec((256, 128), lambda i: (0, 0)),
        ],
        out_specs=pl.BlockSpec((BR, 128), lambda i: (i, 0)),
        out_shape=jax.ShapeDtypeStruct((NP, 128), jnp.float32),
    )(pa, pb, va, vb, b, dis, w)


def _tc_fin(p, v, b3, dis):
    def body(p_ref, v_ref, b_ref, dis_ref, mu_ref, ls_ref):
        o = (p_ref[0] + p_ref[1] - v_ref[...]) * dis_ref[...] + b_ref[...]
        mu_ref[...] = o[:, :64]
        ls_ref[...] = o[:, 64:]

    return pl.pallas_call(
        body,
        grid=(GN,),
        in_specs=[
            pl.BlockSpec((2, BR, 128), lambda i: (0, i, 0)),
            pl.BlockSpec((BR, 128), lambda i: (i, 0)),
            pl.BlockSpec((1, 128), lambda i: (0, 0)),
            pl.BlockSpec((BR, 1), lambda i: (i, 0)),
        ],
        out_specs=[
            pl.BlockSpec((BR, 64), lambda i: (i, 0)),
            pl.BlockSpec((BR, 64), lambda i: (i, 0)),
        ],
        out_shape=[
            jax.ShapeDtypeStruct((NP, 64), jnp.float32),
            jax.ShapeDtypeStruct((NP, 64), jnp.float32),
        ],
    )(p, v, b3, dis)


def kernel(x, edge_index, W1, b1, W2, b2, Wmu, bmu, Wls, bls):
    xp = jnp.pad(x, ((0, NP - N), (0, 0)))
    pad = jnp.full((EP - E,), N, jnp.int32)
    srcp = jnp.concatenate([edge_index[0], pad]).reshape(ERP, K)
    dstp = jnp.concatenate([edge_index[1], pad]).reshape(ERP, K)
    w3 = jnp.concatenate([Wmu, Wls], axis=1)
    b3 = jnp.concatenate([bmu, bls]).reshape(1, 128)

    degp = _deg(dstp).reshape(2, NP, 1)
    dis, v1 = _tc_prep(xp, W1, degp)
    p1 = _agg(v1, srcp, dstp)
    v2a, v2b = _tc_mid1(p1.reshape(2, NP, 128), v1, b1.reshape(1, 128),
                        dis, W2)
    p2a = _agg(v2a, srcp, dstp)
    p2b = _agg(v2b, srcp, dstp)
    v3 = _tc_mid2(p2a.reshape(2, NP, 128), p2b.reshape(2, NP, 128),
                  v2a, v2b, b2.reshape(1, 256), dis, w3)
    p3 = _agg(v3, srcp, dstp)
    mu, logstd = _tc_fin(p3.reshape(2, NP, 128), v3, b3, dis)
    recon = _dec(mu, srcp, dstp)
    return (recon[:E], mu[:N], logstd[:N])
